# baseline (speedup 1.0000x reference)
import functools

import jax
import jax.numpy as jnp
from jax import lax
from jax.experimental import pallas as pl
from jax.experimental.pallas import tpu as pltpu
from jax.experimental.pallas import tpu_sc as plsc

D_MODEL = 4096
N_EXP = 64
TOP_K = 8
TOKENS = 32768

NC = 2
NS = 16
NW = NC * NS
LANES = 16

CHUNKS = (16384, 16384)

BT = 512
HB = BT // 2


def _matmul_body(x_ref, w_ref, o_ref):
    logits = lax.dot_general(
        x_ref[...], w_ref[...],
        dimension_numbers=(((1,), (1,)), ((), ())),
        preferred_element_type=jnp.float32,
    )
    o_ref[...] = jnp.concatenate([logits[:HB], logits[HB:]], axis=1)


def _logits_tc(x, W, off, ch):
    return pl.pallas_call(
        _matmul_body,
        grid=(ch // BT,),
        in_specs=[
            pl.BlockSpec((BT, D_MODEL),
                         lambda i, off=off: (off // BT + i, 0)),
            pl.BlockSpec((N_EXP, D_MODEL), lambda i: (0, 0)),
        ],
        out_specs=pl.BlockSpec((BT // 2, 2 * N_EXP), lambda i: (i, 0)),
        out_shape=jax.ShapeDtypeStruct((ch // 2, 2 * N_EXP), jnp.float32),
    )(x, W)


_mesh = plsc.VectorSubcoreMesh(
    core_axis_name="c", subcore_axis_name="s", num_cores=NC, num_subcores=NS)


@functools.lru_cache(maxsize=None)
def _make_topk_sc(ch):
    tpw = ch // NW
    rpw = tpw // 2
    assert tpw % BT == 0

    @functools.partial(
        pl.kernel,
        mesh=_mesh,
        out_type=[
            jax.ShapeDtypeStruct((ch * TOP_K,), jnp.float32),
            jax.ShapeDtypeStruct((ch * TOP_K,), jnp.int32),
        ],
        scratch_types=[
            pltpu.VMEM((rpw, 2 * N_EXP), jnp.float32),
            pltpu.VMEM((tpw * TOP_K + LANES - TOP_K,), jnp.float32),
            pltpu.VMEM((tpw * TOP_K + LANES - TOP_K,), jnp.int32),
        ],
        compiler_params=pltpu.CompilerParams(
            needs_layout_passes=False, use_tc_tiling_on_sc=False),
    )
    def _topk_sc(logits_hbm, probs_hbm, idx_hbm, lv, pv, iv):
        wid = lax.axis_index("s") * NC + lax.axis_index("c")
        base = wid * tpw
        pltpu.sync_copy(logits_hbm.at[pl.ds(wid * rpw, rpw), :], lv)

        lane = lax.iota(jnp.int32, LANES)
        lo_mask = lane < TOP_K

        def merge(va, ia, vb, ib):
            vbr = lax.rev(vb, (0,))
            ibr = lax.rev(ib, (0,))
            vm = jnp.where(lo_mask, va, vbr)
            im = jnp.where(lo_mask, ia, ibr)
            return plsc.sort_key_val(vm, im, descending=True)

        def one_token(r, h):
            sv = []
            si = []
            for g in range(N_EXP // LANES):
                v = lv[r, pl.ds(h * N_EXP + g * LANES, LANES)]
                s_v, s_i = plsc.sort_key_val(
                    v, lane + g * LANES, descending=True)
                sv.append(s_v)
                si.append(s_i)
            v01, i01 = merge(sv[0], si[0], sv[1], si[1])
            v23, i23 = merge(sv[2], si[2], sv[3], si[3])
            v, i = merge(v01, i01, v23, i23)

            m = lax.reduce_max(v, axes=(0,))
            e = jnp.where(lo_mask, jnp.exp(v - m), 0.0)
            s = lax.reduce_sum(e, axes=(0,))
            p = e / s

            t = (r // HB) * BT + h * HB + lax.rem(r, HB)
            plsc.store_compressed(
                pv.at[pl.ds(t * TOP_K, LANES)], p, mask=lo_mask)
            plsc.store_compressed(
                iv.at[pl.ds(t * TOP_K, LANES)], i, mask=lo_mask)

        @plsc.parallel_loop(0, rpw, unroll=2)
        def body(r):
            one_token(r, 0)
            one_token(r, 1)

        pltpu.sync_copy(pv.at[pl.ds(0, tpw * TOP_K)],
                        probs_hbm.at[pl.ds(base * TOP_K, tpw * TOP_K)])
        pltpu.sync_copy(iv.at[pl.ds(0, tpw * TOP_K)],
                        idx_hbm.at[pl.ds(base * TOP_K, tpw * TOP_K)])

    return _topk_sc


@functools.lru_cache(maxsize=None)
def _make_topk_sc_merge(ch, prev):
    tpw = ch // NW
    rpw = tpw // 2
    cpw = prev * TOP_K // NW
    out_n = (prev + ch) * TOP_K
    assert tpw % BT == 0

    @functools.partial(
        pl.kernel,
        mesh=_mesh,
        out_type=[
            jax.ShapeDtypeStruct((out_n,), jnp.float32),
            jax.ShapeDtypeStruct((out_n,), jnp.int32),
        ],
        scratch_types=[
            pltpu.VMEM((rpw, 2 * N_EXP), jnp.float32),
            pltpu.VMEM((tpw * TOP_K + LANES - TOP_K,), jnp.float32),
            pltpu.VMEM((tpw * TOP_K + LANES - TOP_K,), jnp.int32),
            pltpu.VMEM((cpw,), jnp.float32),
            pltpu.VMEM((cpw,), jnp.int32),
        ],
        compiler_params=pltpu.CompilerParams(
            needs_layout_passes=False, use_tc_tiling_on_sc=False),
    )
    def _topk_sc(logits_hbm, p0_hbm, i0_hbm, probs_hbm, idx_hbm,
                 lv, pv, iv, cpf, cpi):
        wid = lax.axis_index("s") * NC + lax.axis_index("c")
        base = wid * tpw
        pltpu.sync_copy(logits_hbm.at[pl.ds(wid * rpw, rpw), :], lv)

        pltpu.sync_copy(p0_hbm.at[pl.ds(wid * cpw, cpw)], cpf)
        pltpu.sync_copy(cpf, probs_hbm.at[pl.ds(wid * cpw, cpw)])
        pltpu.sync_copy(i0_hbm.at[pl.ds(wid * cpw, cpw)], cpi)
        pltpu.sync_copy(cpi, idx_hbm.at[pl.ds(wid * cpw, cpw)])

        lane = lax.iota(jnp.int32, LANES)
        lo_mask = lane < TOP_K

        def merge(va, ia, vb, ib):
            vbr = lax.rev(vb, (0,))
            ibr = lax.rev(ib, (0,))
            vm = jnp.where(lo_mask, va, vbr)
            im = jnp.where(lo_mask, ia, ibr)
            return plsc.sort_key_val(vm, im, descending=True)

        def one_token(r, h):
            sv = []
            si = []
            for g in range(N_EXP // LANES):
                v = lv[r, pl.ds(h * N_EXP + g * LANES, LANES)]
                s_v, s_i = plsc.sort_key_val(
                    v, lane + g * LANES, descending=True)
                sv.append(s_v)
                si.append(s_i)
            v01, i01 = merge(sv[0], si[0], sv[1], si[1])
            v23, i23 = merge(sv[2], si[2], sv[3], si[3])
            v, i = merge(v01, i01, v23, i23)

            m = lax.reduce_max(v, axes=(0,))
            e = jnp.where(lo_mask, jnp.exp(v - m), 0.0)
            s = lax.reduce_sum(e, axes=(0,))
            p = e / s

            t = (r // HB) * BT + h * HB + lax.rem(r, HB)
            plsc.store_compressed(
                pv.at[pl.ds(t * TOP_K, LANES)], p, mask=lo_mask)
            plsc.store_compressed(
                iv.at[pl.ds(t * TOP_K, LANES)], i, mask=lo_mask)

        @plsc.parallel_loop(0, rpw, unroll=2)
        def body(r):
            one_token(r, 0)
            one_token(r, 1)

        tail = prev * TOP_K
        pltpu.sync_copy(pv.at[pl.ds(0, tpw * TOP_K)],
                        probs_hbm.at[pl.ds(tail + base * TOP_K, tpw * TOP_K)])
        pltpu.sync_copy(iv.at[pl.ds(0, tpw * TOP_K)],
                        idx_hbm.at[pl.ds(tail + base * TOP_K, tpw * TOP_K)])

    return _topk_sc


def kernel(x, W):
    ch0, ch1 = CHUNKS
    logits0 = _logits_tc(x, W, 0, ch0)
    p0, i0 = _make_topk_sc(ch0)(logits0)
    logits1 = _logits_tc(x, W, ch0, ch1)
    pf, if_ = _make_topk_sc_merge(ch1, ch0)(logits1, p0, i0)
    return (pf.reshape(TOKENS, TOP_K), if_.reshape(TOKENS, TOP_K))


# speedup vs baseline: 1.0066x; 1.0066x over previous
import functools

import jax
import jax.numpy as jnp
from jax import lax
from jax.experimental import pallas as pl
from jax.experimental.pallas import tpu as pltpu
from jax.experimental.pallas import tpu_sc as plsc

D_MODEL = 4096
N_EXP = 64
TOP_K = 8
TOKENS = 32768

NC = 2
NS = 16
NW = NC * NS
LANES = 16

CHUNKS = (16384, 16384)

BT = 512
HB = BT // 2


def _matmul_body(x_ref, w_ref, o_ref):
    logits = lax.dot_general(
        x_ref[...], w_ref[...],
        dimension_numbers=(((1,), (1,)), ((), ())),
        preferred_element_type=jnp.float32,
    )
    o_ref[...] = jnp.concatenate([logits[:HB], logits[HB:]], axis=1)


def _logits_tc(x, W, off, ch):
    return pl.pallas_call(
        _matmul_body,
        grid=(ch // BT,),
        in_specs=[
            pl.BlockSpec((BT, D_MODEL),
                         lambda i, off=off: (off // BT + i, 0)),
            pl.BlockSpec((N_EXP, D_MODEL), lambda i: (0, 0)),
        ],
        out_specs=pl.BlockSpec((BT // 2, 2 * N_EXP), lambda i: (i, 0)),
        out_shape=jax.ShapeDtypeStruct((ch // 2, 2 * N_EXP), jnp.float32),
    )(x, W)


_mesh = plsc.VectorSubcoreMesh(
    core_axis_name="c", subcore_axis_name="s", num_cores=NC, num_subcores=NS)


def _routing_loop(lv, pv2, iv2, tpw):
    lane = lax.iota(jnp.int32, LANES)
    lo_mask = lane < TOP_K
    shift = jnp.maximum(lane - TOP_K, 0)

    def merge(va, ia, vb, ib):
        vbr = lax.rev(vb, (0,))
        ibr = lax.rev(ib, (0,))
        vm = jnp.where(lo_mask, va, vbr)
        im = jnp.where(lo_mask, ia, ibr)
        return plsc.sort_key_val(vm, im, descending=True)

    def one_token(r, h):
        sv = []
        si = []
        for g in range(N_EXP // LANES):
            v = lv[r, pl.ds(h * N_EXP + g * LANES, LANES)]
            s_v, s_i = plsc.sort_key_val(
                v, lane + g * LANES, descending=True)
            sv.append(s_v)
            si.append(s_i)
        v01, i01 = merge(sv[0], si[0], sv[1], si[1])
        v23, i23 = merge(sv[2], si[2], sv[3], si[3])
        v, i = merge(v01, i01, v23, i23)

        m = lax.reduce_max(v, axes=(0,))
        e = jnp.where(lo_mask, jnp.exp(v - m), 0.0)
        s = lax.reduce_sum(e, axes=(0,))
        return e / s, i

    @plsc.parallel_loop(0, tpw // 2, unroll=2)
    def body(p):
        t = 2 * p
        b = t // BT
        tt = lax.rem(t, BT)
        h = tt // HB
        r0 = b * HB + lax.rem(tt, HB)
        p_a, i_a = one_token(r0, h)
        p_b, i_b = one_token(r0 + 1, h)
        p_bs = p_b.at[shift].get(mode="promise_in_bounds")
        i_bs = i_b.at[shift].get(mode="promise_in_bounds")
        pv2[p, :] = jnp.where(lo_mask, p_a, p_bs)
        iv2[p, :] = jnp.where(lo_mask, i_a, i_bs)


@functools.lru_cache(maxsize=None)
def _make_topk_sc(ch):
    tpw = ch // NW
    rpw = tpw // 2
    assert tpw % BT == 0

    @functools.partial(
        pl.kernel,
        mesh=_mesh,
        out_type=[
            jax.ShapeDtypeStruct((ch // 2, 2 * TOP_K), jnp.float32),
            jax.ShapeDtypeStruct((ch // 2, 2 * TOP_K), jnp.int32),
        ],
        scratch_types=[
            pltpu.VMEM((rpw, 2 * N_EXP), jnp.float32),
            pltpu.VMEM((rpw, 2 * TOP_K), jnp.float32),
            pltpu.VMEM((rpw, 2 * TOP_K), jnp.int32),
        ],
        compiler_params=pltpu.CompilerParams(
            needs_layout_passes=False, use_tc_tiling_on_sc=False),
    )
    def _topk_sc(logits_hbm, probs_hbm, idx_hbm, lv, pv2, iv2):
        wid = lax.axis_index("s") * NC + lax.axis_index("c")
        pltpu.sync_copy(logits_hbm.at[pl.ds(wid * rpw, rpw), :], lv)
        _routing_loop(lv, pv2, iv2, tpw)
        pltpu.sync_copy(pv2, probs_hbm.at[pl.ds(wid * rpw, rpw), :])
        pltpu.sync_copy(iv2, idx_hbm.at[pl.ds(wid * rpw, rpw), :])

    return _topk_sc


@functools.lru_cache(maxsize=None)
def _make_topk_sc_merge(ch, prev):
    tpw = ch // NW
    rpw = tpw // 2
    cpr = prev // 2 // NW
    assert tpw % BT == 0

    @functools.partial(
        pl.kernel,
        mesh=_mesh,
        out_type=[
            jax.ShapeDtypeStruct(((prev + ch) // 2, 2 * TOP_K), jnp.float32),
            jax.ShapeDtypeStruct(((prev + ch) // 2, 2 * TOP_K), jnp.int32),
        ],
        scratch_types=[
            pltpu.VMEM((rpw, 2 * N_EXP), jnp.float32),
            pltpu.VMEM((rpw, 2 * TOP_K), jnp.float32),
            pltpu.VMEM((rpw, 2 * TOP_K), jnp.int32),
            pltpu.VMEM((cpr, 2 * TOP_K), jnp.float32),
            pltpu.VMEM((cpr, 2 * TOP_K), jnp.int32),
        ],
        compiler_params=pltpu.CompilerParams(
            needs_layout_passes=False, use_tc_tiling_on_sc=False),
    )
    def _topk_sc(logits_hbm, p0_hbm, i0_hbm, probs_hbm, idx_hbm,
                 lv, pv2, iv2, cpf, cpi):
        wid = lax.axis_index("s") * NC + lax.axis_index("c")
        pltpu.sync_copy(logits_hbm.at[pl.ds(wid * rpw, rpw), :], lv)

        pltpu.sync_copy(p0_hbm.at[pl.ds(wid * cpr, cpr), :], cpf)
        pltpu.sync_copy(cpf, probs_hbm.at[pl.ds(wid * cpr, cpr), :])
        pltpu.sync_copy(i0_hbm.at[pl.ds(wid * cpr, cpr), :], cpi)
        pltpu.sync_copy(cpi, idx_hbm.at[pl.ds(wid * cpr, cpr), :])

        _routing_loop(lv, pv2, iv2, tpw)

        tail = prev // 2
        pltpu.sync_copy(pv2, probs_hbm.at[pl.ds(tail + wid * rpw, rpw), :])
        pltpu.sync_copy(iv2, idx_hbm.at[pl.ds(tail + wid * rpw, rpw), :])

    return _topk_sc


def kernel(x, W):
    ch0, ch1 = CHUNKS
    logits0 = _logits_tc(x, W, 0, ch0)
    p0, i0 = _make_topk_sc(ch0)(logits0)
    logits1 = _logits_tc(x, W, ch0, ch1)
    pf, if_ = _make_topk_sc_merge(ch1, ch0)(logits1, p0, i0)
    return (pf.reshape(TOKENS, TOP_K), if_.reshape(TOKENS, TOP_K))


# speedup vs baseline: 1.1745x; 1.1667x over previous
import functools

import jax
import jax.numpy as jnp
from jax import lax
from jax.experimental import pallas as pl
from jax.experimental.pallas import tpu as pltpu
from jax.experimental.pallas import tpu_sc as plsc

D_MODEL = 4096
N_EXP = 64
TOP_K = 8
TOKENS = 32768

NC = 2
NS = 16
NW = NC * NS
LANES = 16

CHUNKS = (16384, 16384)

BT = 512
HB = BT // 2


def _matmul_body(x_ref, w_ref, o_ref):
    logits = lax.dot_general(
        x_ref[...], w_ref[...],
        dimension_numbers=(((1,), (1,)), ((), ())),
        preferred_element_type=jnp.float32,
    )
    o_ref[...] = jnp.concatenate([logits[:HB], logits[HB:]], axis=1)


def _logits_tc(x, W, off, ch):
    return pl.pallas_call(
        _matmul_body,
        grid=(ch // BT,),
        in_specs=[
            pl.BlockSpec((BT, D_MODEL),
                         lambda i, off=off: (off // BT + i, 0)),
            pl.BlockSpec((N_EXP, D_MODEL), lambda i: (0, 0)),
        ],
        out_specs=pl.BlockSpec((BT // 2, 2 * N_EXP), lambda i: (i, 0)),
        out_shape=jax.ShapeDtypeStruct((ch // 2, 2 * N_EXP), jnp.float32),
    )(x, W)


_mesh = plsc.VectorSubcoreMesh(
    core_axis_name="c", subcore_axis_name="s", num_cores=NC, num_subcores=NS)


def _routing_loop(lv, pvT, ivT, tpw):
    lane = lax.iota(jnp.int32, LANES)
    lo_mask = lane < TOP_K
    rows = jnp.bitwise_and(lane, TOP_K - 1)

    def merge(va, ia, vb, ib):
        vbr = lax.rev(vb, (0,))
        ibr = lax.rev(ib, (0,))
        vm = jnp.where(lo_mask, va, vbr)
        im = jnp.where(lo_mask, ia, ibr)
        return plsc.sort_key_val(vm, im, descending=True)

    def one_token(r, h):
        sv = []
        si = []
        for g in range(N_EXP // LANES):
            v = lv[r, pl.ds(h * N_EXP + g * LANES, LANES)]
            s_v, s_i = plsc.sort_key_val(
                v, lane + g * LANES, descending=True)
            sv.append(s_v)
            si.append(s_i)
        v01, i01 = merge(sv[0], si[0], sv[1], si[1])
        v23, i23 = merge(sv[2], si[2], sv[3], si[3])
        v, i = merge(v01, i01, v23, i23)

        m = lax.reduce_max(v, axes=(0,))
        e = jnp.where(lo_mask, jnp.exp(v - m), 0.0)
        s = lax.reduce_sum(e, axes=(0,))
        p = e / s

        t = (r // HB) * BT + h * HB + lax.rem(r, HB)
        cols = jnp.full((LANES,), t, jnp.int32)
        plsc.store_scatter(pvT, [rows, cols], p, mask=lo_mask)
        plsc.store_scatter(ivT, [rows, cols], i, mask=lo_mask)

    @plsc.parallel_loop(0, tpw // 2, unroll=2)
    def body(r):
        one_token(r, 0)
        one_token(r, 1)


@functools.lru_cache(maxsize=None)
def _make_topk_sc(ch):
    tpw = ch // NW
    rpw = tpw // 2
    assert tpw % BT == 0

    @functools.partial(
        pl.kernel,
        mesh=_mesh,
        out_type=[
            jax.ShapeDtypeStruct((TOP_K, ch), jnp.float32),
            jax.ShapeDtypeStruct((TOP_K, ch), jnp.int32),
        ],
        scratch_types=[
            pltpu.VMEM((rpw, 2 * N_EXP), jnp.float32),
            pltpu.VMEM((TOP_K, tpw), jnp.float32),
            pltpu.VMEM((TOP_K, tpw), jnp.int32),
        ],
        compiler_params=pltpu.CompilerParams(
            needs_layout_passes=False, use_tc_tiling_on_sc=False),
    )
    def _topk_sc(logits_hbm, probs_hbm, idx_hbm, lv, pvT, ivT):
        wid = lax.axis_index("s") * NC + lax.axis_index("c")
        pltpu.sync_copy(logits_hbm.at[pl.ds(wid * rpw, rpw), :], lv)
        _routing_loop(lv, pvT, ivT, tpw)
        pltpu.sync_copy(pvT, probs_hbm.at[:, pl.ds(wid * tpw, tpw)])
        pltpu.sync_copy(ivT, idx_hbm.at[:, pl.ds(wid * tpw, tpw)])

    return _topk_sc


@functools.lru_cache(maxsize=None)
def _make_topk_sc_merge(ch, prev):
    tpw = ch // NW
    rpw = tpw // 2
    cpt = prev // NW
    assert tpw % BT == 0

    @functools.partial(
        pl.kernel,
        mesh=_mesh,
        out_type=[
            jax.ShapeDtypeStruct((TOP_K, prev + ch), jnp.float32),
            jax.ShapeDtypeStruct((TOP_K, prev + ch), jnp.int32),
        ],
        scratch_types=[
            pltpu.VMEM((rpw, 2 * N_EXP), jnp.float32),
            pltpu.VMEM((TOP_K, tpw), jnp.float32),
            pltpu.VMEM((TOP_K, tpw), jnp.int32),
            pltpu.VMEM((TOP_K, cpt), jnp.float32),
            pltpu.VMEM((TOP_K, cpt), jnp.int32),
        ],
        compiler_params=pltpu.CompilerParams(
            needs_layout_passes=False, use_tc_tiling_on_sc=False),
    )
    def _topk_sc(logits_hbm, p0_hbm, i0_hbm, probs_hbm, idx_hbm,
                 lv, pvT, ivT, cpf, cpi):
        wid = lax.axis_index("s") * NC + lax.axis_index("c")
        pltpu.sync_copy(logits_hbm.at[pl.ds(wid * rpw, rpw), :], lv)

        pltpu.sync_copy(p0_hbm.at[:, pl.ds(wid * cpt, cpt)], cpf)
        pltpu.sync_copy(cpf, probs_hbm.at[:, pl.ds(wid * cpt, cpt)])
        pltpu.sync_copy(i0_hbm.at[:, pl.ds(wid * cpt, cpt)], cpi)
        pltpu.sync_copy(cpi, idx_hbm.at[:, pl.ds(wid * cpt, cpt)])

        _routing_loop(lv, pvT, ivT, tpw)

        pltpu.sync_copy(pvT, probs_hbm.at[:, pl.ds(prev + wid * tpw, tpw)])
        pltpu.sync_copy(ivT, idx_hbm.at[:, pl.ds(prev + wid * tpw, tpw)])

    return _topk_sc


def kernel(x, W):
    ch0, ch1 = CHUNKS
    logits0 = _logits_tc(x, W, 0, ch0)
    p0, i0 = _make_topk_sc(ch0)(logits0)
    logits1 = _logits_tc(x, W, ch0, ch1)
    pT, iT = _make_topk_sc_merge(ch1, ch0)(logits1, p0, i0)
    return (pT.T, iT.T)
